# Initial kernel scaffold; baseline (speedup 1.0000x reference)
import jax
import jax.numpy as jnp
from jax.experimental import pallas as pl


def kernel(obj_loc, obj_ske, obj_type, h, edge_index, W_e1, b_e1, W_e2, b_e2, Wa1, Wa2, Ww, W_ih, W_hh, b_ih, b_hh, W_out, b_out):
    raise NotImplementedError("write your pallas kernel here")


# speedup vs baseline: 4.9349x; 4.9349x over previous
import functools

import jax
import jax.numpy as jnp
from jax import lax
from jax.experimental import pallas as pl
from jax.experimental.pallas import tpu as pltpu
from jax.experimental.pallas import tpu_sc as plsc

N = 10000
E = 320000
D = 128
NC, NS = 2, 16
NW = NC * NS
EPW = E // NW
C = 80
NCHUNK = EPW // C
NPAD = 10240
RPT = NPAD // NS
BN = 2000


def _proj_body(ske, typ, loc, h, w1s, w1t, w1l, b1, w2, b2,
               mqd, mqs, mpd, mps, tdq, tsq, tdp, tsp):
    e1 = ske[...] @ w1s[...] + typ[...] @ w1t[...] + loc[...] @ w1l[...]
    e1 = jnp.maximum(e1 + b1[...], 0.0)
    emb = jnp.maximum(e1 @ w2[...] + b2[...], 0.0)
    hb = h[...]
    tdq[...] = hb @ mqd[...]
    tsq[...] = hb @ mqs[...]
    tdp[...] = emb @ mpd[...]
    tsp[...] = emb @ mps[...]


_proj = pl.pallas_call(
    _proj_body,
    grid=(N // BN,),
    in_specs=[
        pl.BlockSpec((BN, D), lambda i: (i, 0)),
        pl.BlockSpec((BN, 16), lambda i: (i, 0)),
        pl.BlockSpec((BN, D), lambda i: (i, 0)),
        pl.BlockSpec((BN, D), lambda i: (i, 0)),
        pl.BlockSpec((D, D), lambda i: (0, 0)),
        pl.BlockSpec((16, D), lambda i: (0, 0)),
        pl.BlockSpec((D, D), lambda i: (0, 0)),
        pl.BlockSpec((1, D), lambda i: (0, 0)),
        pl.BlockSpec((D, D), lambda i: (0, 0)),
        pl.BlockSpec((1, D), lambda i: (0, 0)),
        pl.BlockSpec((D, D), lambda i: (0, 0)),
        pl.BlockSpec((D, D), lambda i: (0, 0)),
        pl.BlockSpec((D, D), lambda i: (0, 0)),
        pl.BlockSpec((D, D), lambda i: (0, 0)),
    ],
    out_specs=[pl.BlockSpec((BN, D), lambda i: (i, 0)) for _ in range(4)],
    out_shape=[jax.ShapeDtypeStruct((N, D), jnp.float32) for _ in range(4)],
)


_mesh = plsc.VectorSubcoreMesh(core_axis_name="c", subcore_axis_name="s",
                               num_cores=NC, num_subcores=NS)

NG = C // 16
ND = NPAD // D


@functools.partial(
    pl.kernel,
    out_type=[
        jax.ShapeDtypeStruct((NC, NPAD, D), jnp.float32),
        jax.ShapeDtypeStruct((NC, ND, D), jnp.float32),
    ],
    mesh=_mesh,
    scratch_types=[
        pltpu.VMEM((C,), jnp.int32),
        pltpu.VMEM((C,), jnp.int32),
        pltpu.VMEM((C,), jnp.int32),
        pltpu.VMEM((C, D), jnp.float32),
        pltpu.VMEM((C, D), jnp.float32),
        pltpu.VMEM((C, D), jnp.float32),
        pltpu.VMEM((C, D), jnp.float32),
        pltpu.VMEM((C,), jnp.float32),
        pltpu.VMEM((D,), jnp.float32),
        pltpu.VMEM_SHARED((NPAD, D), jnp.float32),
        pltpu.VMEM_SHARED((ND, D), jnp.float32),
        pltpu.SemaphoreType.DMA,
    ],
    compiler_params=pltpu.CompilerParams(needs_layout_passes=False),
)
def _edge_pass(tdq_hbm, tsq_hbm, tdp_hbm, tsp_hbm, dst_hbm, src_hbm,
               wa2_hbm, zero_hbm, out_msg, out_den,
               idx_d, idx_s, idx_2, rows_a, rows_b, obuf, dbuf, exs_v, wa2_v,
               msg_tab, den_tab, sem):
    cid = lax.axis_index("c")
    sid = lax.axis_index("s")
    wid = cid * NS + sid

    pltpu.sync_copy(zero_hbm.at[pl.ds(sid * (NPAD // NS), NPAD // NS)],
                    msg_tab.at[pl.ds(sid * (NPAD // NS), NPAD // NS)])

    @pl.when(sid == 0)
    def _zero_den():
        pltpu.sync_copy(zero_hbm.at[pl.ds(0, ND)], den_tab)

    pltpu.sync_copy(wa2_hbm, wa2_v)

    zero16 = jnp.zeros((16,), jnp.float32)

    def zero_body(i, c):
        for j in range(8):
            dbuf[i, pl.ds(16 * j, 16)] = zero16
        return c
    lax.fori_loop(0, C, zero_body, 0)

    plsc.subcore_barrier()

    wa2v = [wa2_v[pl.ds(16 * j, 16)] for j in range(8)]
    lane = lax.iota(jnp.int32, 16)
    lane0 = lane == 0

    def chunk_body(k, carry):
        base = wid * EPW + k * C
        pltpu.sync_copy(dst_hbm.at[pl.ds(base, C)], idx_d)
        pltpu.sync_copy(src_hbm.at[pl.ds(base, C)], idx_s)

        rows_l, cols_l = [], []
        for g in range(NG):
            idxv = idx_d[pl.ds(16 * g, 16)]
            idx_2[pl.ds(16 * g, 16)] = lax.shift_right_logical(idxv, 7)
            rows_l.append(lane + (16 * g))
            cols_l.append(idxv & (D - 1))

        pltpu.async_copy(tdq_hbm.at[idx_d], rows_a, sem).wait()
        pltpu.async_copy(tsq_hbm.at[idx_s], rows_b, sem).wait()

        def logit_body(e, c2):
            acc = jnp.zeros((16,), jnp.float32)
            for j in range(8):
                t = rows_a[e, pl.ds(16 * j, 16)] + rows_b[e, pl.ds(16 * j, 16)]
                t = jnp.where(t > 0.0, t, 0.01 * t)
                acc = acc + t * wa2v[j]
            logit = jnp.sum(acc)
            exv = jnp.exp(jnp.full((16,), logit, jnp.float32))
            plsc.store_scatter(exs_v, [jnp.full((16,), e, jnp.int32)], exv,
                               mask=lane0)
            return c2

        lax.fori_loop(0, C, logit_body, 0)

        for g in range(NG):
            ex16 = exs_v[pl.ds(16 * g, 16)]
            plsc.store_scatter(dbuf, [rows_l[g], cols_l[g]], ex16)

        pltpu.async_copy(tdp_hbm.at[idx_d], rows_a, sem).wait()
        pltpu.async_copy(tsp_hbm.at[idx_s], rows_b, sem).wait()

        def msg_body(e, c2):
            ex1 = plsc.load_gather(exs_v, [jnp.full((16,), e, jnp.int32)])
            for j in range(8):
                u = rows_a[e, pl.ds(16 * j, 16)] + rows_b[e, pl.ds(16 * j, 16)]
                obuf[e, pl.ds(16 * j, 16)] = jnp.maximum(u, 0.0) * ex1
            return c2

        lax.fori_loop(0, C, msg_body, 0)

        pltpu.sync_copy(obuf, msg_tab.at[idx_d], add=True)
        pltpu.sync_copy(dbuf, den_tab.at[idx_2], add=True)

        for g in range(NG):
            plsc.store_scatter(dbuf, [rows_l[g], cols_l[g]], zero16)
        return carry

    lax.fori_loop(0, NCHUNK, chunk_body, 0)
    plsc.subcore_barrier()
    pltpu.sync_copy(msg_tab.at[pl.ds(sid * (NPAD // NS), NPAD // NS)],
                    out_msg.at[cid, pl.ds(sid * (NPAD // NS), NPAD // NS)])

    @pl.when(sid == 0)
    def _copy_den():
        pltpu.sync_copy(den_tab, out_den.at[cid])


def _update_body(msg, den, h, wih, whh, bih, bhh, wout, bout, out):
    a = msg[0] + msg[1]
    d = den[0] + den[1]
    agg = a / (d + 1e-9)
    gi = agg @ wih[...] + bih[...]
    gh = h[...] @ whh[...] + bhh[...]
    r = jax.nn.sigmoid(gi[:, :D] + gh[:, :D])
    z = jax.nn.sigmoid(gi[:, D:2 * D] + gh[:, D:2 * D])
    n = jnp.tanh(gi[:, 2 * D:] + r * gh[:, 2 * D:])
    hn = (1.0 - z) * n + z * h[...]
    out[...] = jnp.maximum(hn @ wout[...] + bout[...], 0.0)


_update = pl.pallas_call(
    _update_body,
    grid=(N // BN,),
    in_specs=[
        pl.BlockSpec((NC, BN, D), lambda i: (0, i, 0)),
        pl.BlockSpec((NC, BN, 1), lambda i: (0, i, 0)),
        pl.BlockSpec((BN, D), lambda i: (i, 0)),
        pl.BlockSpec((D, 3 * D), lambda i: (0, 0)),
        pl.BlockSpec((D, 3 * D), lambda i: (0, 0)),
        pl.BlockSpec((1, 3 * D), lambda i: (0, 0)),
        pl.BlockSpec((1, 3 * D), lambda i: (0, 0)),
        pl.BlockSpec((D, D), lambda i: (0, 0)),
        pl.BlockSpec((1, D), lambda i: (0, 0)),
    ],
    out_specs=pl.BlockSpec((BN, D), lambda i: (i, 0)),
    out_shape=jax.ShapeDtypeStruct((N, D), jnp.float32),
)


def kernel(obj_loc, obj_ske, obj_type, h, edge_index, W_e1, b_e1, W_e2, b_e2,
           Wa1, Wa2, Ww, W_ih, W_hh, b_ih, b_hh, W_out, b_out):
    ei = edge_index.astype(jnp.int32)
    src, dst = ei[0], ei[1]

    w1 = W_e1.T
    w1s, w1t, w1l = w1[:D], w1[D:D + 16], w1[D + 16:]
    b1 = b_e1.reshape(1, D)
    b2 = b_e2.reshape(1, D)
    mqd = (Wa1[:, :D] + Wa1[:, D:]).T
    mqs = -Wa1[:, D:].T
    mpd = (Ww[:, :D] + Ww[:, D:]).T
    mps = -Ww[:, D:].T

    tdq, tsq, tdp, tsp = _proj(obj_ske, obj_type, obj_loc, h, w1s, w1t, w1l,
                               b1, W_e2.T, b2, mqd, mqs, mpd, mps)

    wa2 = Wa2.reshape(D)
    zeros = jnp.zeros((NPAD, D), jnp.float32)
    msg, den = _edge_pass(tdq, tsq, tdp, tsp, dst, src, wa2, zeros)
    den2d = den.reshape(NC, NPAD, 1)

    return _update(msg, den2d, h, W_ih.T, W_hh.T, b_ih.reshape(1, 3 * D),
                   b_hh.reshape(1, 3 * D), W_out.T, b_out.reshape(1, D))
